# baseline (speedup 1.0000x reference)
import functools

import jax
import jax.numpy as jnp
from jax import lax
from jax.experimental import pallas as pl
from jax.experimental.pallas import tpu as pltpu
from jax.experimental.pallas import tpu_sc as plsc

N = 6400000
R = 100000
NC = 2
NS = 16
NW = NC * NS
HALF = N
SPT = HALF // NW
C = 2000
SUB = 2000
NCHUNK = SPT // C
ACC = 16 * 18752
SLICE = ACC // NS
MAXW = 12288
W3 = 3 * MAXW
FB = 768
UNROLL = 125
STRIDE = SUB // 16


def _sc_partials(rp, gp, bp, w, idx):
    mesh = plsc.VectorSubcoreMesh(core_axis_name="c", subcore_axis_name="s")

    @functools.partial(
        pl.kernel,
        mesh=mesh,
        compiler_params=pltpu.CompilerParams(needs_layout_passes=False),
        out_type=jax.ShapeDtypeStruct((NC * ACC,), jnp.float32),
        scratch_types=[
            pltpu.VMEM((C,), jnp.float32),
            pltpu.VMEM((C,), jnp.float32),
            pltpu.VMEM((C,), jnp.float32),
            pltpu.VMEM((C,), jnp.float32),
            pltpu.VMEM((C,), jnp.float32),
            pltpu.VMEM((C,), jnp.float32),
            pltpu.VMEM((C,), jnp.float32),
            pltpu.VMEM((C,), jnp.float32),
            pltpu.VMEM((C,), jnp.int32),
            pltpu.VMEM((C,), jnp.int32),
            pltpu.VMEM((3 * SUB,), jnp.float32),
            pltpu.VMEM((3 * SUB,), jnp.int32),
            pltpu.VMEM((16,), jnp.int32),
            pltpu.VMEM((FB,), jnp.int32),
            pltpu.VMEM((W3,), jnp.float32),
            pltpu.VMEM_SHARED((ACC,), jnp.float32),
            pltpu.SemaphoreType.DMA,
            pltpu.SemaphoreType.DMA,
        ],
    )
    def k(r_hbm, g_hbm, b_hbm, w_hbm, idx_hbm, out_hbm, r0_v, r1_v, g0_v,
          g1_v, b0_v, b1_v, w0_v, w1_v, i0_v, i1_v, val_v, tgt_v, if_v, tb_v,
          acc2, acc_sh, sem0, sem1):
        c = lax.axis_index("c")
        s = lax.axis_index("s")
        wid = c * NS + s

        lane = lax.iota(jnp.int32, 16)
        zeros16 = jnp.zeros((16,), jnp.float32)
        sems = (sem0, sem1)
        slots = (((r_hbm, r0_v), (g_hbm, g0_v), (b_hbm, b0_v),
                  (w_hbm, w0_v), (idx_hbm, i0_v)),
                 ((r_hbm, r1_v), (g_hbm, g1_v), (b_hbm, b1_v),
                  (w_hbm, w1_v), (idx_hbm, i1_v)))

        def z2(i, carry):
            acc2[pl.ds(16 * i, 16)] = zeros16
            return carry
        lax.fori_loop(0, W3 // 16, z2, 0)

        HS = SLICE // 2
        def zbody(i, carry):
            val_v[pl.ds(16 * i, 16)] = zeros16
            return carry
        lax.fori_loop(0, HS // 16, zbody, 0)
        pltpu.sync_copy(val_v.at[pl.ds(0, HS)],
                        acc_sh.at[pl.ds(s * SLICE, HS)])
        pltpu.sync_copy(val_v.at[pl.ds(0, HS)],
                        acc_sh.at[pl.ds(s * SLICE + HS, HS)])
        plsc.subcore_barrier()

        base = wid * SPT
        pltpu.sync_copy(idx_hbm.at[pl.ds(base, 16)], if_v)
        base_ray = jnp.min(if_v[...])
        base3 = base_ray * 3

        def issue(kk, slot):
            s0 = base + kk * C
            for hbm, vbuf in slots[slot]:
                pltpu.async_copy(hbm.at[pl.ds(s0, C)], vbuf, sems[slot])

        def drain(slot):
            for hbm, vbuf in slots[slot]:
                pltpu.make_async_copy(hbm.at[pl.ds(0, C)], vbuf,
                                      sems[slot]).wait()

        def process(slot):
            rv, gv, bv, wv, iv = (p[1] for p in slots[slot])
            last_ray = jnp.max(iv[pl.ds(C - 16, 16)])
            in_window = (last_ray - base_ray) * 3 + 2 < W3

            def fast():
                for h in range(C // SUB):
                    ls = lane * STRIDE + h * SUB

                    def grp(j, icarry):
                        for u in range(UNROLL):
                            gi = ls + (UNROLL * j + u)
                            w16 = plsc.load_gather(wv, [gi])
                            t0 = plsc.load_gather(iv, [gi]) * 3 - base3
                            plsc.addupdate_scatter(
                                acc2, [t0], plsc.load_gather(rv, [gi]) * w16)
                            plsc.addupdate_scatter(
                                acc2, [t0 + 1], plsc.load_gather(gv, [gi]) * w16)
                            plsc.addupdate_scatter(
                                acc2, [t0 + 2], plsc.load_gather(bv, [gi]) * w16)
                        return icarry
                    lax.fori_loop(0, STRIDE // UNROLL, grp, 0)

            def slow():
                for h in range(C // SUB):
                    hb = h * SUB

                    def grp(j, icarry):
                        off = 16 * j
                        w16 = wv[pl.ds(hb + off, 16)]
                        t0 = iv[pl.ds(hb + off, 16)] * 3
                        val_v[pl.ds(off, 16)] = rv[pl.ds(hb + off, 16)] * w16
                        tgt_v[pl.ds(off, 16)] = t0
                        val_v[pl.ds(SUB + off, 16)] = gv[pl.ds(hb + off, 16)] * w16
                        tgt_v[pl.ds(SUB + off, 16)] = t0 + 1
                        val_v[pl.ds(2 * SUB + off, 16)] = bv[pl.ds(hb + off, 16)] * w16
                        tgt_v[pl.ds(2 * SUB + off, 16)] = t0 + 2
                        return icarry
                    lax.fori_loop(0, SUB // 16, grp, 0)
                    pltpu.sync_copy(val_v, acc_sh.at[tgt_v], add=True)

            lax.cond(in_window, fast, slow)

        issue(0, 0)

        def outer(i, carry):
            for b in range(2):
                kk = 2 * i + b
                drain(b)

                @pl.when(kk + 1 < NCHUNK)
                def _prefetch():
                    issue(kk + 1, 1 - b)

                process(b)
            return carry
        lax.fori_loop(0, NCHUNK // 2, outer, 0)

        tile_last = jnp.max(i1_v[pl.ds(C - 16, 16)])
        span3 = jnp.minimum((tile_last - base_ray + 1) * 3, W3)
        nblk = (span3 + FB - 1) // FB

        def fblk(b, carry):
            off = FB * b

            def bld(g, icarry):
                tb_v[pl.ds(16 * g, 16)] = jnp.minimum(
                    base3 + off + 16 * g + lane, ACC - 1)
                return icarry
            lax.fori_loop(0, FB // 16, bld, 0)
            pltpu.sync_copy(acc2.at[pl.ds(off, FB)], acc_sh.at[tb_v],
                            add=True)
            return carry
        lax.fori_loop(0, nblk, fblk, 0)

        plsc.subcore_barrier()
        for h in range(2):
            pltpu.sync_copy(acc_sh.at[pl.ds(s * SLICE + h * HS, HS)],
                            val_v.at[pl.ds(0, HS)])
            pltpu.sync_copy(val_v.at[pl.ds(0, HS)],
                            out_hbm.at[pl.ds(c * ACC + s * SLICE + h * HS, HS)])

    return k(rp, gp, bp, w, idx)


def _merge_kernel(p_ref, o_ref):
    o_ref[...] = jnp.clip(p_ref[0] + p_ref[1], 0.0, 1.0)


def kernel(rgb, weights, ray_indices, num_rays):
    rp = rgb[:, 0]
    gp = rgb[:, 1]
    bp = rgb[:, 2]
    w = weights.reshape(-1)
    idx = ray_indices.astype(jnp.int32)

    partials = _sc_partials(rp, gp, bp, w, idx)
    p = partials.reshape(NC, ACC // 128, 128)
    merged = pl.pallas_call(
        _merge_kernel,
        out_shape=jax.ShapeDtypeStruct((ACC // 128, 128), jnp.float32),
    )(p)
    return merged.reshape(-1)[: 3 * R].reshape(R, 3)


# speedup vs baseline: 1.2822x; 1.2822x over previous
import functools

import jax
import jax.numpy as jnp
from jax import lax
from jax.experimental import pallas as pl
from jax.experimental.pallas import tpu as pltpu
from jax.experimental.pallas import tpu_sc as plsc

N = 6400000
R = 100000
NC = 2
NS = 16
NW = NC * NS
HALF = N
SPT = HALF // NW
C = 2000
SUB = 2000
NCHUNK = SPT // C
ACC = 16 * 18752
SLICE = ACC // NS
MAXW = 12288
W3 = 3 * MAXW
FB = 768
UNROLL = 25
STRIDE = SUB // 16


def _sc_partials(rp, gp, bp, w, idx):
    mesh = plsc.VectorSubcoreMesh(core_axis_name="c", subcore_axis_name="s")

    @functools.partial(
        pl.kernel,
        mesh=mesh,
        compiler_params=pltpu.CompilerParams(needs_layout_passes=False),
        out_type=jax.ShapeDtypeStruct((NC * ACC,), jnp.float32),
        scratch_types=[
            pltpu.VMEM((C,), jnp.float32),
            pltpu.VMEM((C,), jnp.float32),
            pltpu.VMEM((C,), jnp.float32),
            pltpu.VMEM((C,), jnp.float32),
            pltpu.VMEM((C,), jnp.float32),
            pltpu.VMEM((C,), jnp.float32),
            pltpu.VMEM((C,), jnp.float32),
            pltpu.VMEM((C,), jnp.float32),
            pltpu.VMEM((C,), jnp.int32),
            pltpu.VMEM((C,), jnp.int32),
            pltpu.VMEM((3 * SUB,), jnp.float32),
            pltpu.VMEM((3 * SUB,), jnp.int32),
            pltpu.VMEM((16,), jnp.int32),
            pltpu.VMEM((FB,), jnp.int32),
            pltpu.VMEM((W3,), jnp.float32),
            pltpu.VMEM_SHARED((ACC,), jnp.float32),
            pltpu.SemaphoreType.DMA,
            pltpu.SemaphoreType.DMA,
        ],
    )
    def k(r_hbm, g_hbm, b_hbm, w_hbm, idx_hbm, out_hbm, r0_v, r1_v, g0_v,
          g1_v, b0_v, b1_v, w0_v, w1_v, i0_v, i1_v, val_v, tgt_v, if_v, tb_v,
          acc2, acc_sh, sem0, sem1):
        c = lax.axis_index("c")
        s = lax.axis_index("s")
        wid = c * NS + s

        lane = lax.iota(jnp.int32, 16)
        zeros16 = jnp.zeros((16,), jnp.float32)
        sems = (sem0, sem1)
        slots = (((r_hbm, r0_v), (g_hbm, g0_v), (b_hbm, b0_v),
                  (w_hbm, w0_v), (idx_hbm, i0_v)),
                 ((r_hbm, r1_v), (g_hbm, g1_v), (b_hbm, b1_v),
                  (w_hbm, w1_v), (idx_hbm, i1_v)))

        def z2(i, carry):
            acc2[pl.ds(16 * i, 16)] = zeros16
            return carry
        lax.fori_loop(0, W3 // 16, z2, 0)

        HS = SLICE // 2
        def zbody(i, carry):
            val_v[pl.ds(16 * i, 16)] = zeros16
            return carry
        lax.fori_loop(0, HS // 16, zbody, 0)
        pltpu.sync_copy(val_v.at[pl.ds(0, HS)],
                        acc_sh.at[pl.ds(s * SLICE, HS)])
        pltpu.sync_copy(val_v.at[pl.ds(0, HS)],
                        acc_sh.at[pl.ds(s * SLICE + HS, HS)])
        plsc.subcore_barrier()

        base = wid * SPT
        pltpu.sync_copy(idx_hbm.at[pl.ds(base, 16)], if_v)
        base_ray = jnp.min(if_v[...])
        base3 = base_ray * 3

        def issue(kk, slot):
            s0 = base + kk * C
            for hbm, vbuf in slots[slot]:
                pltpu.async_copy(hbm.at[pl.ds(s0, C)], vbuf, sems[slot])

        def drain(slot):
            for hbm, vbuf in slots[slot]:
                pltpu.make_async_copy(hbm.at[pl.ds(0, C)], vbuf,
                                      sems[slot]).wait()

        def process(slot):
            rv, gv, bv, wv, iv = (p[1] for p in slots[slot])
            last_ray = jnp.max(iv[pl.ds(C - 16, 16)])
            in_window = (last_ray - base_ray) * 3 + 2 < W3

            def fast():
                for h in range(C // SUB):
                    ls = lane * STRIDE + h * SUB

                    def grp(j, icarry):
                        for u in range(UNROLL):
                            gi = ls + (UNROLL * j + u)
                            w16 = plsc.load_gather(wv, [gi])
                            t0 = plsc.load_gather(iv, [gi]) * 3 - base3
                            plsc.addupdate_scatter(
                                acc2, [t0], plsc.load_gather(rv, [gi]) * w16)
                            plsc.addupdate_scatter(
                                acc2, [t0 + 1], plsc.load_gather(gv, [gi]) * w16)
                            plsc.addupdate_scatter(
                                acc2, [t0 + 2], plsc.load_gather(bv, [gi]) * w16)
                        return icarry
                    lax.fori_loop(0, STRIDE // UNROLL, grp, 0)

            def slow():
                for h in range(C // SUB):
                    hb = h * SUB

                    def grp(j, icarry):
                        off = 16 * j
                        w16 = wv[pl.ds(hb + off, 16)]
                        t0 = iv[pl.ds(hb + off, 16)] * 3
                        val_v[pl.ds(off, 16)] = rv[pl.ds(hb + off, 16)] * w16
                        tgt_v[pl.ds(off, 16)] = t0
                        val_v[pl.ds(SUB + off, 16)] = gv[pl.ds(hb + off, 16)] * w16
                        tgt_v[pl.ds(SUB + off, 16)] = t0 + 1
                        val_v[pl.ds(2 * SUB + off, 16)] = bv[pl.ds(hb + off, 16)] * w16
                        tgt_v[pl.ds(2 * SUB + off, 16)] = t0 + 2
                        return icarry
                    lax.fori_loop(0, SUB // 16, grp, 0)
                    pltpu.sync_copy(val_v, acc_sh.at[tgt_v], add=True)

            lax.cond(in_window, fast, slow)

        issue(0, 0)

        def outer(i, carry):
            for b in range(2):
                kk = 2 * i + b
                drain(b)

                @pl.when(kk + 1 < NCHUNK)
                def _prefetch():
                    issue(kk + 1, 1 - b)

                process(b)
            return carry
        lax.fori_loop(0, NCHUNK // 2, outer, 0)

        tile_last = jnp.max(i1_v[pl.ds(C - 16, 16)])
        span3 = jnp.minimum((tile_last - base_ray + 1) * 3, W3)
        nblk = (span3 + FB - 1) // FB

        def fblk(b, carry):
            off = FB * b

            def bld(g, icarry):
                tb_v[pl.ds(16 * g, 16)] = jnp.minimum(
                    base3 + off + 16 * g + lane, ACC - 1)
                return icarry
            lax.fori_loop(0, FB // 16, bld, 0)
            pltpu.sync_copy(acc2.at[pl.ds(off, FB)], acc_sh.at[tb_v],
                            add=True)
            return carry
        lax.fori_loop(0, nblk, fblk, 0)

        plsc.subcore_barrier()
        for h in range(2):
            pltpu.sync_copy(acc_sh.at[pl.ds(s * SLICE + h * HS, HS)],
                            val_v.at[pl.ds(0, HS)])
            pltpu.sync_copy(val_v.at[pl.ds(0, HS)],
                            out_hbm.at[pl.ds(c * ACC + s * SLICE + h * HS, HS)])

    return k(rp, gp, bp, w, idx)


def _merge_kernel(p_ref, o_ref):
    o_ref[...] = jnp.clip(p_ref[0] + p_ref[1], 0.0, 1.0)


def kernel(rgb, weights, ray_indices, num_rays):
    rp = rgb[:, 0]
    gp = rgb[:, 1]
    bp = rgb[:, 2]
    w = weights.reshape(-1)
    idx = ray_indices.astype(jnp.int32)

    partials = _sc_partials(rp, gp, bp, w, idx)
    p = partials.reshape(NC, ACC // 128, 128)
    merged = pl.pallas_call(
        _merge_kernel,
        out_shape=jax.ShapeDtypeStruct((ACC // 128, 128), jnp.float32),
    )(p)
    return merged.reshape(-1)[: 3 * R].reshape(R, 3)
